# baseline (speedup 1.0000x reference)
import functools

import jax
import jax.numpy as jnp
from jax import lax
from jax.experimental import pallas as pl
from jax.experimental.pallas import tpu as pltpu
from jax.experimental.pallas import tpu_sc as plsc

NUM_CORES = 2
NUM_SUBCORES = 16
CHUNK = 128


def kernel(input_ids, table):
    batch, seq = input_ids.shape
    vocab, hidden = table.shape
    num_idx = batch * seq
    flat_idx = input_ids.reshape(num_idx)

    num_workers = NUM_CORES * NUM_SUBCORES
    per_worker = num_idx // num_workers
    n_chunks = per_worker // CHUNK

    mesh = plsc.VectorSubcoreMesh(core_axis_name="c", subcore_axis_name="s")

    @functools.partial(
        pl.kernel,
        mesh=mesh,
        out_type=jax.ShapeDtypeStruct((num_idx, hidden), table.dtype),
        scratch_types=[
            pltpu.VMEM((CHUNK,), jnp.int32),
            pltpu.VMEM((CHUNK, hidden), jnp.float32),
            pltpu.SemaphoreType.DMA,
        ],
        compiler_params=pltpu.CompilerParams(use_tc_tiling_on_sc=False),
    )
    def gather_kernel(table_hbm, idx_hbm, out_hbm, idx_v, rows_v, sem):
        wid = lax.axis_index("s") * NUM_CORES + lax.axis_index("c")
        base_w = wid * per_worker

        @pl.loop(0, n_chunks)
        def _(t):
            base = base_w + t * CHUNK
            pltpu.sync_copy(idx_hbm.at[pl.ds(base, CHUNK)], idx_v)
            pltpu.async_copy(table_hbm.at[idx_v], rows_v, sem).wait()
            pltpu.sync_copy(rows_v, out_hbm.at[pl.ds(base, CHUNK)])

    out = gather_kernel(table, flat_idx)
    return out.reshape(batch, seq, hidden)


# speedup vs baseline: 1.1971x; 1.1971x over previous
import functools

import jax
import jax.numpy as jnp
from jax import lax
from jax.experimental import pallas as pl
from jax.experimental.pallas import tpu as pltpu
from jax.experimental.pallas import tpu_sc as plsc

NUM_CORES = 2
NUM_SUBCORES = 16
CHUNK = 128
NBUF = 4


def kernel(input_ids, table):
    batch, seq = input_ids.shape
    vocab, hidden = table.shape
    num_idx = batch * seq
    flat_idx = input_ids.reshape(num_idx)

    num_workers = NUM_CORES * NUM_SUBCORES
    per_worker = num_idx // num_workers
    n_chunks = per_worker // CHUNK
    n_outer = n_chunks // NBUF

    mesh = plsc.VectorSubcoreMesh(core_axis_name="c", subcore_axis_name="s")

    @functools.partial(
        pl.kernel,
        mesh=mesh,
        out_type=jax.ShapeDtypeStruct((num_idx, hidden), table.dtype),
        scratch_types=[
            pltpu.VMEM((per_worker,), jnp.int32),
        ]
        + [pltpu.VMEM((CHUNK, hidden), jnp.float32)] * NBUF
        + [pltpu.SemaphoreType.DMA] * (2 * NBUF)
        + [pltpu.SemaphoreType.DMA],
        compiler_params=pltpu.CompilerParams(use_tc_tiling_on_sc=False),
    )
    def gather_kernel(table_hbm, idx_hbm, out_hbm, idx_all, *rest):
        bufs = rest[:NBUF]
        g_sem = rest[NBUF:2 * NBUF]
        s_sem = rest[2 * NBUF:3 * NBUF]
        idx_sem = rest[3 * NBUF]

        wid = lax.axis_index("s") * NUM_CORES + lax.axis_index("c")
        base_w = wid * per_worker

        pltpu.async_copy(
            idx_hbm.at[pl.ds(base_w, per_worker)], idx_all, idx_sem).wait()

        def idx_slice(t):
            return idx_all.at[pl.ds(t * CHUNK, CHUNK)]

        def out_slice(t):
            return out_hbm.at[pl.ds(base_w + t * CHUNK, CHUNK)]

        def gather(t, b):
            pltpu.make_async_copy(
                table_hbm.at[idx_slice(t)], bufs[b], g_sem[b]).start()

        def store(t, b):
            pltpu.make_async_copy(bufs[b], out_slice(t), s_sem[b]).start()

        for b in range(NBUF):
            gather(b, b)

        @pl.loop(0, n_outer)
        def _(k):
            for b in range(NBUF):
                t = k * NBUF + b
                pltpu.make_async_copy(
                    table_hbm.at[idx_slice(t)], bufs[b], g_sem[b]).wait()
                store(t, b)

                @pl.when(k < n_outer - 1)
                def _():
                    pltpu.make_async_copy(bufs[b], out_slice(t), s_sem[b]).wait()
                    gather(t + NBUF, b)

        for b in range(NBUF):
            t = (n_outer - 1) * NBUF + b
            pltpu.make_async_copy(bufs[b], out_slice(t), s_sem[b]).wait()

    out = gather_kernel(table, flat_idx)
    return out.reshape(batch, seq, hidden)
